# Initial kernel scaffold; baseline (speedup 1.0000x reference)
import jax
import jax.numpy as jnp
from jax.experimental import pallas as pl


def kernel(proj_main, proj_ema, label_main, label_ema, patch_num):
    raise NotImplementedError("write your pallas kernel here")


# speedup vs baseline: 1.5394x; 1.5394x over previous
import functools

import jax
import jax.numpy as jnp
from jax.experimental import pallas as pl
from jax.experimental.pallas import tpu as pltpu

TEMP = 0.1
EPS = 1e-8
LOG2E = 1.4426950408889634
BM = 256


def _loss_block_kernel(e_ref, at_ref, et_ref, out_ref):
    scale = jnp.float32(LOG2E / TEMP)
    a = at_ref[...] * scale
    s = jnp.dot(e_ref[...], a, preferred_element_type=jnp.float32)
    pos = jnp.sum(a * et_ref[...], axis=0, keepdims=True)
    m = jnp.maximum(jnp.max(s, axis=0, keepdims=True), pos)
    d_neg = jnp.sum(jnp.exp2(s - m), axis=0, keepdims=True)
    p = jnp.exp2(pos - m)
    ratio = p / (d_neg + p + jnp.float32(EPS))
    out_ref[0] = -jnp.log(ratio + jnp.float32(EPS))


@functools.partial(jax.jit, static_argnames=())
def _contrastive_loss(proj_main, proj_ema):
    b, c, H, W = proj_main.shape
    N = b * H * W
    at = proj_main.transpose(1, 0, 2, 3).reshape(c, N)
    e = proj_ema.transpose(0, 2, 3, 1).reshape(N, c)
    et = proj_ema.transpose(1, 0, 2, 3).reshape(c, N)
    grid = (N // BM,)
    losses = pl.pallas_call(
        _loss_block_kernel,
        grid=grid,
        in_specs=[
            pl.BlockSpec((N, c), lambda i: (0, 0)),
            pl.BlockSpec((c, BM), lambda i: (0, i)),
            pl.BlockSpec((c, BM), lambda i: (0, i)),
        ],
        out_specs=pl.BlockSpec((1, 1, BM), lambda i: (i, 0, 0)),
        out_shape=jax.ShapeDtypeStruct((N // BM, 1, BM), jnp.float32),
        compiler_params=pltpu.CompilerParams(
            dimension_semantics=("parallel",),
            vmem_limit_bytes=100 * 1024 * 1024,
        ),
    )(e, at, et)
    return jnp.mean(losses)


def kernel(proj_main, proj_ema, label_main, label_ema, patch_num):
    return _contrastive_loss(proj_main, proj_ema)
